# baseline (speedup 1.0000x reference)
import functools

import jax
import jax.numpy as jnp
from jax import lax
from jax.experimental import pallas as pl
from jax.experimental.pallas import tpu as pltpu
from jax.experimental.pallas import tpu_sc as plsc

NC = 2
NS = 16
LANES = 16
NBUF = 4


def _make_kernel(B, T, D, V, P):
    NW = NC * NS
    BW = B // NW
    W2 = 2 * D
    PH = ((T // 2 + 7) // 8) * 8
    mesh = plsc.VectorSubcoreMesh(
        core_axis_name="c", subcore_axis_name="s", num_cores=NC, num_subcores=NS
    )

    @functools.partial(
        pl.kernel,
        mesh=mesh,
        compiler_params=pltpu.CompilerParams(
            use_tc_tiling_on_sc=True, needs_layout_passes=False),
        out_type=jax.ShapeDtypeStruct((T, D, B), jnp.float32),
        scratch_types=[
            pltpu.VMEM((T, BW), jnp.int32),
            pltpu.VMEM((NBUF, BW), jnp.int32),
            pltpu.VMEM((PH, W2), jnp.float32),
            pltpu.VMEM((NBUF, BW, W2), jnp.float32),
            pltpu.VMEM((D, BW), jnp.float32),
            pltpu.VMEM((D, BW), jnp.float32),
            pltpu.SemaphoreType.DMA,
            pltpu.SemaphoreType.DMA,
            pltpu.SemaphoreType.DMA,
            pltpu.SemaphoreType.DMA,
            pltpu.SemaphoreType.DMA,
            pltpu.SemaphoreType.DMA,
        ],
    )
    def ker(ids_hbm, tab_hbm, pos_hbm, out_hbm, idx_v, idxh_v, pos_v,
            gath_v, ot0, ot1, g0, g1, g2, g3, o0, o1):
        ot = (ot0, ot1)
        gsem = (g0, g1, g2, g3)
        osem = (o0, o1)
        wid = lax.axis_index("s") * NC + lax.axis_index("c")
        col = wid * BW

        pltpu.sync_copy(ids_hbm.at[:, pl.ds(col, BW)], idx_v)
        pltpu.sync_copy(pos_hbm.at[pl.ds(0, PH)], pos_v)

        iota16 = lax.iota(jnp.int32, LANES)

        def fire_gather(t, s):
            for k in range(BW // LANES):
                sl = pl.ds(k * LANES, LANES)
                idxh_v[s, sl] = lax.shift_right_logical(idx_v[t, sl], 1)
            pltpu.async_copy(
                tab_hbm.at[idxh_v.at[s]], gath_v.at[s], gsem[s]
            )

        def drain_g(s):
            pltpu.make_async_copy(
                tab_hbm.at[pl.ds(0, BW)], gath_v.at[s], gsem[s]).wait()

        def drain_o(s):
            pltpu.make_async_copy(
                tab_hbm.at[pl.ds(0, D)], ot[s], osem[s]).wait()

        def compute(t, gs, os):
            vraw = [idx_v[t, pl.ds(j * LANES, LANES)]
                    for j in range(BW // LANES)]
            cols = [lax.mul(lax.rem(v, 2), D) for v in vraw]
            rows = [iota16 + (j * LANES) for j in range(BW // LANES)]
            th = lax.shift_right_logical(t, 1)
            toff = lax.mul(lax.rem(t, 2), D)

            @plsc.parallel_loop(0, D // LANES, 1, unroll=2)
            def d_block(db):
                d0 = db * LANES
                ps_vec = pos_v[th, pl.ds(toff + d0, LANES)]
                colsb = [c + d0 for c in cols]
                for dd in range(LANES):
                    ps = ps_vec[dd]
                    for j in range(BW // LANES):
                        vals = plsc.load_gather(
                            gath_v.at[gs], [rows[j], colsb[j] + dd])
                        ot[os][d0 + dd, pl.ds(j * LANES, LANES)] = vals + ps

        def fire_out(t, s):
            pltpu.async_copy(ot[s], out_hbm.at[t, :, pl.ds(col, BW)], osem[s])

        for t0 in range(NBUF - 1):
            fire_gather(t0, t0)

        def quad_body(tq, carry):
            for u in range(NBUF):
                t = tq * NBUF + u
                gs = u
                os = u % 2
                drain_g(gs)

                @pl.when(t + (NBUF - 1) < T)
                def _():
                    fire_gather(t + (NBUF - 1), (u + NBUF - 1) % NBUF)

                @pl.when(t >= 2)
                def _():
                    drain_o(os)

                compute(t, gs, os)
                fire_out(t, os)
            return carry

        lax.fori_loop(0, T // NBUF, quad_body, 0)
        drain_o(0)
        drain_o(1)

    return ker


def kernel(input_ids, semantic_table, pos_table):
    B, T = input_ids.shape
    V, D = semantic_table.shape
    P = pos_table.shape[0]
    NW = NC * NS
    assert B % (NW * 128) == 0 and D == 64 and T % NBUF == 0 and V % 2 == 0

    ker = _make_kernel(B, T, D, V, P)
    out_tdb = ker(
        jnp.swapaxes(input_ids, 0, 1),
        semantic_table.reshape(V // 2, 2 * D),
        pos_table.reshape(P // 2, 2 * D),
    )
    return jnp.transpose(out_tdb, (2, 0, 1))


# speedup vs baseline: 1.2474x; 1.2474x over previous
import functools

import jax
import jax.numpy as jnp
from jax import lax
from jax.experimental import pallas as pl
from jax.experimental.pallas import tpu as pltpu
from jax.experimental.pallas import tpu_sc as plsc

NC = 2
NS = 16
LANES = 16


def _make_kernel(N, V, D, P, T, B, n_per_w, C, G):
    n_chunks = n_per_w // C
    BW = n_per_w // T
    mesh = plsc.VectorSubcoreMesh(
        core_axis_name="c", subcore_axis_name="s", num_cores=NC, num_subcores=NS
    )

    @functools.partial(
        pl.kernel,
        mesh=mesh,
        compiler_params=pltpu.CompilerParams(
            use_tc_tiling_on_sc=False, needs_layout_passes=False),
        out_type=jax.ShapeDtypeStruct((N, D), jnp.float32),
        scratch_types=[
            pltpu.VMEM((C, D), jnp.float32),
            pltpu.VMEM((T, BW), jnp.int32),
            pltpu.VMEM((n_per_w,), jnp.int32),
            pltpu.VMEM((C, D), jnp.float32),
            pltpu.VMEM((C, D), jnp.float32),
            pltpu.SemaphoreType.DMA,
            pltpu.SemaphoreType.DMA,
            pltpu.SemaphoreType.DMA,
            pltpu.SemaphoreType.DMA,
        ],
    )
    def ker(ids_hbm, tab_hbm, pos_hbm, out_hbm, pos_v, idst_v, idx_v,
            rows0, rows1, gsem0, gsem1, osem0, osem1):
        rows = (rows0, rows1)
        gsem = (gsem0, gsem1)
        osem = (osem0, osem1)
        wid = lax.axis_index("s") * NC + lax.axis_index("c")
        base = wid * n_per_w
        pltpu.sync_copy(ids_hbm.at[:, pl.ds(wid * BW, BW)], idst_v)
        for k in range(C // T):
            pltpu.sync_copy(pos_hbm.at[pl.ds(0, T)], pos_v.at[pl.ds(k * T, T)])

        iota16 = lax.iota(jnp.int32, LANES)
        tcols = [(lax.rem(iota16 + (k * LANES), T),
                  lax.div(iota16 + (k * LANES), T))
                 for k in range(2 * T // LANES)]

        @plsc.parallel_loop(0, BW // 2, 1)
        def trans_body(bp):
            for k, (tv, cd) in enumerate(tcols):
                vals = plsc.load_gather(idst_v, [tv, cd + bp * 2])
                idx_v[pl.ds(bp * (2 * T) + k * LANES, LANES)] = vals

        def fire_gathers(g, b):
            for j in range(C // G):
                pltpu.async_copy(
                    tab_hbm.at[idx_v.at[pl.ds(g * C + j * G, G)]],
                    rows[b].at[pl.ds(j * G, G)],
                    gsem[b],
                )

        def drain(sem, b):
            pltpu.make_async_copy(tab_hbm.at[pl.ds(0, C)], rows[b], sem).wait()

        fire_gathers(0, 0)

        def pair_body(gp, carry):
            for b in (0, 1):
                g = gp * 2 + b
                o = 1 - b
                drain(gsem[b], b)

                @plsc.parallel_loop(0, C, 1, unroll=8)
                def row_body(r):
                    for j in range(D // LANES):
                        sl = pl.ds(j * LANES, LANES)
                        rows[b][r, sl] = rows[b][r, sl] + pos_v[r, sl]

                @pl.when(jnp.logical_and(g >= 1, g + 1 < n_chunks))
                def _():
                    drain(osem[o], o)

                @pl.when(g + 1 < n_chunks)
                def _():
                    fire_gathers(g + 1, o)

                pltpu.async_copy(
                    rows[b], out_hbm.at[pl.ds(base + g * C, C)], osem[b]
                )
            return carry

        lax.fori_loop(0, n_chunks // 2, pair_body, 0)
        drain(osem[0], 0)
        drain(osem[1], 1)

    return ker


def kernel(input_ids, semantic_table, pos_table):
    B, T = input_ids.shape
    V, D = semantic_table.shape
    P = pos_table.shape[0]
    N = B * T
    NW = NC * NS
    n_per_w = N // NW
    C = 2 * T
    G = 80
    assert N % NW == 0 and n_per_w % C == 0 and C % G == 0 and D % LANES == 0
    assert (n_per_w // C) % 2 == 0 and G % 8 == 0
    assert n_per_w % T == 0 and (2 * T) % LANES == 0 and (n_per_w // T) % 2 == 0

    ker = _make_kernel(N, V, D, P, T, B, n_per_w, C, G)
    out_flat = ker(jnp.swapaxes(input_ids, 0, 1), semantic_table, pos_table)
    return out_flat.reshape(B, T, D)
